# baseline (speedup 1.0000x reference)
import functools

import jax
import jax.numpy as jnp
from jax import lax
from jax.experimental import pallas as pl
from jax.experimental.pallas import tpu as pltpu
from jax.experimental.pallas import tpu_sc as plsc

N = 10000
H = 128
E = 320000
NC = 2
NS = 16
NW = NC * NS
EWP = 10240
EP = EWP * NW
K = 64
NCHUNK = EWP // K
KD = 128
NCHUNK_D = EWP // KD
NBUF = 2
ACC_R = N + 8
DEG_R = N + 2000

_mesh = plsc.VectorSubcoreMesh(
    core_axis_name="c", subcore_axis_name="s", num_cores=NC, num_subcores=NS)


@functools.partial(
    pl.kernel,
    out_type=jax.ShapeDtypeStruct((NC * N,), jnp.float32),
    mesh=_mesh,
    scratch_types=[
        pltpu.VMEM((NCHUNK_D, KD), jnp.int32),
        pltpu.VMEM((KD,), jnp.float32),
        pltpu.VMEM((2000,), jnp.float32),
        pltpu.VMEM_SHARED((DEG_R,), jnp.float32),
    ],
)
def _sc_degree(dst_hbm, ones_hbm, zeros_hbm, out_hbm,
               idx_v, ones_v, stage_v, deg_sh):
    c = lax.axis_index("c")
    s = lax.axis_index("s")
    wid = c * NS + s

    @pl.when(s == 0)
    def _():
        pltpu.sync_copy(zeros_hbm, stage_v)
        for t in range(DEG_R // 2000):
            pltpu.sync_copy(stage_v, deg_sh.at[pl.ds(t * 2000, 2000)])

    pltpu.sync_copy(dst_hbm.at[wid], idx_v)
    pltpu.sync_copy(ones_hbm, ones_v)
    plsc.subcore_barrier()

    def body(j, carry):
        pltpu.sync_copy(ones_v, deg_sh.at[idx_v.at[j]], add=True)
        return carry

    lax.fori_loop(0, NCHUNK_D, body, 0)
    plsc.subcore_barrier()

    @pl.when(s == 0)
    def _():
        for t in range(N // 2000):
            pltpu.sync_copy(deg_sh.at[pl.ds(t * 2000, 2000)], stage_v)
            pltpu.sync_copy(stage_v, out_hbm.at[pl.ds(c * N + t * 2000, 2000)])


@functools.partial(
    pl.kernel,
    out_type=jax.ShapeDtypeStruct((NC * N, H), jnp.float32),
    mesh=_mesh,
    scratch_types=[
        pltpu.VMEM((EWP,), jnp.int32),
        pltpu.VMEM((NCHUNK, K), jnp.int32),
        pltpu.VMEM((NBUF, K, H), jnp.float32),
        pltpu.VMEM_SHARED((ACC_R, H), jnp.float32),
        [pltpu.SemaphoreType.DMA] * NBUF,
    ],
)
def _sc_scatter(y_hbm, src_hbm, dst_hbm, zrows_hbm, out_hbm,
                src_v, dst_v, rows_v, acc_sh, sems):
    c = lax.axis_index("c")
    s = lax.axis_index("s")
    wid = c * NS + s

    @pl.when(s < NS - 1)
    def _():
        pltpu.sync_copy(zrows_hbm, acc_sh.at[pl.ds(s * 632, 632)])

    @pl.when(s == NS - 1)
    def _():
        pltpu.sync_copy(zrows_hbm.at[pl.ds(0, 528)],
                        acc_sh.at[pl.ds(15 * 632, 528)])

    pltpu.sync_copy(src_hbm.at[wid], src_v)
    pltpu.sync_copy(dst_hbm.at[wid], dst_v)
    plsc.subcore_barrier()

    for i in range(NBUF):
        pltpu.async_copy(y_hbm.at[src_v.at[pl.ds(i * K, K)]],
                         rows_v.at[i], sems[i])

    def _step(j, i, issue_next):
        pltpu.make_async_copy(y_hbm.at[src_v.at[pl.ds(j * K, K)]],
                              rows_v.at[i], sems[i]).wait()
        pltpu.sync_copy(rows_v.at[i], acc_sh.at[dst_v.at[j]], add=True)
        if issue_next:
            pltpu.async_copy(y_hbm.at[src_v.at[pl.ds((j + NBUF) * K, K)]],
                             rows_v.at[i], sems[i])

    n_main = (NCHUNK - NBUF) // NBUF

    def body(g, carry):
        for i in range(NBUF):
            _step(g * NBUF + i, i, True)
        return carry

    lax.fori_loop(0, n_main, body, 0)
    for j in range(n_main * NBUF, NCHUNK):
        _step(j, j % NBUF, j + NBUF < NCHUNK)
    plsc.subcore_barrier()

    @pl.when(s < NS - 1)
    def _():
        pltpu.sync_copy(acc_sh.at[pl.ds(s * 632, 632)],
                        out_hbm.at[pl.ds(c * N + s * 632, 632)])

    @pl.when(s == NS - 1)
    def _():
        pltpu.sync_copy(acc_sh.at[pl.ds(15 * 632, 520)],
                        out_hbm.at[pl.ds(c * N + 15 * 632, 520)])


BR = 2000


def _tc_prescale_body(x_ref, w_ref, b_ref, d0_ref, d1_ref,
                      y_ref, r_ref, dis_ref):
    xw = jnp.dot(x_ref[...], w_ref[...], preferred_element_type=jnp.float32)
    degt = d0_ref[...] + d1_ref[...] + 1.0
    dis = lax.rsqrt(degt)
    y_ref[...] = xw * dis
    r_ref[...] = xw / degt + x_ref[...] + b_ref[...]
    dis_ref[...] = dis


def _tc_prescale(x, W, b2, d0, d1):
    grid = (N // BR,)
    return pl.pallas_call(
        _tc_prescale_body,
        grid=grid,
        in_specs=[
            pl.BlockSpec((BR, H), lambda i: (i, 0)),
            pl.BlockSpec((H, H), lambda i: (0, 0)),
            pl.BlockSpec((1, H), lambda i: (0, 0)),
            pl.BlockSpec((BR, 1), lambda i: (i, 0)),
            pl.BlockSpec((BR, 1), lambda i: (i, 0)),
        ],
        out_specs=[
            pl.BlockSpec((BR, H), lambda i: (i, 0)),
            pl.BlockSpec((BR, H), lambda i: (i, 0)),
            pl.BlockSpec((BR, 1), lambda i: (i, 0)),
        ],
        out_shape=[
            jax.ShapeDtypeStruct((N, H), jnp.float32),
            jax.ShapeDtypeStruct((N, H), jnp.float32),
            jax.ShapeDtypeStruct((N, 1), jnp.float32),
        ],
    )(x, W, b2, d0, d1)


def _tc_finish_body(p0_ref, p1_ref, r_ref, dis_ref, o_ref):
    h = dis_ref[...] * (p0_ref[...] + p1_ref[...]) + r_ref[...]
    mean = jnp.mean(h, axis=1, keepdims=True)
    cent = h - mean
    var = jnp.mean(cent * cent, axis=1, keepdims=True)
    o_ref[...] = cent * lax.rsqrt(var + 1e-5)


def _tc_finish(p0, p1, r, dis):
    grid = (N // BR,)
    return pl.pallas_call(
        _tc_finish_body,
        grid=grid,
        in_specs=[
            pl.BlockSpec((BR, H), lambda i: (i, 0)),
            pl.BlockSpec((BR, H), lambda i: (i, 0)),
            pl.BlockSpec((BR, H), lambda i: (i, 0)),
            pl.BlockSpec((BR, 1), lambda i: (i, 0)),
        ],
        out_specs=pl.BlockSpec((BR, H), lambda i: (i, 0)),
        out_shape=jax.ShapeDtypeStruct((N, H), jnp.float32),
    )(p0, p1, r, dis)


def kernel(x, edge_index, batch, W, b):
    src = edge_index[0].astype(jnp.int32)
    dst = edge_index[1].astype(jnp.int32)
    npad = EP - E
    src_p = jnp.concatenate([src, jnp.zeros((npad,), jnp.int32)])
    dst_p = jnp.concatenate([dst, jnp.full((npad,), N, jnp.int32)])
    src_p = src_p.reshape(NW, EWP)
    dst3 = dst_p.reshape(NW, NCHUNK, K)
    dstd = dst_p.reshape(NW, NCHUNK_D, KD)

    ones_k = jnp.ones((KD,), jnp.float32)
    zeros_2k = jnp.zeros((2000,), jnp.float32)
    zrows = jnp.zeros((632, H), jnp.float32)

    deg = _sc_degree(dstd, ones_k, zeros_2k)
    d0 = deg[:N].reshape(N, 1)
    d1 = deg[N:].reshape(N, 1)

    y, r, dis = _tc_prescale(x, W, b.reshape(1, H), d0, d1)

    acc = _sc_scatter(y, src_p, dst3, zrows)

    return _tc_finish(acc[:N], acc[N:], r, dis)


# speedup vs baseline: 2.3397x; 2.3397x over previous
import functools

import jax
import jax.numpy as jnp
from jax import lax
from jax.experimental import pallas as pl
from jax.experimental.pallas import tpu as pltpu
from jax.experimental.pallas import tpu_sc as plsc

N = 10000
H = 128
E = 320000
NC = 2
NS = 16
NW = NC * NS
EWP = 10240
EP = EWP * NW
K = 64
NCHUNK = EWP // K
KD = 128
NCHUNK_D = EWP // KD
NBUF = 2
ACC_R = N + 128
DEG_R = N + 2000

_mesh = plsc.VectorSubcoreMesh(
    core_axis_name="c", subcore_axis_name="s", num_cores=NC, num_subcores=NS)


@functools.partial(
    pl.kernel,
    out_type=jax.ShapeDtypeStruct((NC * N,), jnp.float32),
    mesh=_mesh,
    scratch_types=[
        pltpu.VMEM((NCHUNK_D, KD), jnp.int32),
        pltpu.VMEM((KD,), jnp.float32),
        pltpu.VMEM((2000,), jnp.float32),
        pltpu.VMEM_SHARED((DEG_R,), jnp.float32),
    ],
)
def _sc_degree(dst_hbm, ones_hbm, zeros_hbm, out_hbm,
               idx_v, ones_v, stage_v, deg_sh):
    c = lax.axis_index("c")
    s = lax.axis_index("s")
    wid = c * NS + s

    @pl.when(s == 0)
    def _():
        pltpu.sync_copy(zeros_hbm, stage_v)
        for t in range(DEG_R // 2000):
            pltpu.sync_copy(stage_v, deg_sh.at[pl.ds(t * 2000, 2000)])

    pltpu.sync_copy(dst_hbm.at[wid], idx_v)
    pltpu.sync_copy(ones_hbm, ones_v)
    plsc.subcore_barrier()

    def body(j, carry):
        pltpu.sync_copy(ones_v, deg_sh.at[idx_v.at[j]], add=True)
        return carry

    lax.fori_loop(0, NCHUNK_D, body, 0)
    plsc.subcore_barrier()

    @pl.when(s == 0)
    def _():
        for t in range(N // 2000):
            pltpu.sync_copy(deg_sh.at[pl.ds(t * 2000, 2000)], stage_v)
            pltpu.sync_copy(stage_v, out_hbm.at[pl.ds(c * N + t * 2000, 2000)])


@functools.partial(
    pl.kernel,
    out_type=jax.ShapeDtypeStruct((NC * N, H), jnp.float32),
    mesh=_mesh,
    scratch_types=[
        pltpu.VMEM((EWP,), jnp.int32),
        pltpu.VMEM((NCHUNK, K), jnp.int32),
        pltpu.VMEM((NBUF, K, H), jnp.float32),
        pltpu.VMEM_SHARED((ACC_R, H), jnp.float32),
        [pltpu.SemaphoreType.DMA] * NBUF,
    ],
)
def _sc_scatter(y_hbm, src_hbm, dst_hbm, zrows_hbm, out_hbm,
                src_v, dst_v, rows_v, acc_sh, sems):
    c = lax.axis_index("c")
    s = lax.axis_index("s")
    wid = c * NS + s

    @pl.when(s < NS - 1)
    def _():
        pltpu.sync_copy(zrows_hbm.at[pl.ds(0, 632)],
                        acc_sh.at[pl.ds(s * 632, 632)])

    @pl.when(s == NS - 1)
    def _():
        pltpu.sync_copy(zrows_hbm, acc_sh.at[pl.ds(15 * 632, 648)])

    pltpu.sync_copy(src_hbm.at[wid], src_v)
    pltpu.sync_copy(dst_hbm.at[wid], dst_v)
    plsc.subcore_barrier()

    for i in range(NBUF):
        pltpu.async_copy(y_hbm.at[src_v.at[pl.ds(i * K, K)]],
                         rows_v.at[i], sems[i])

    def _step(j, i, issue_next):
        pltpu.make_async_copy(y_hbm.at[src_v.at[pl.ds(j * K, K)]],
                              rows_v.at[i], sems[i]).wait()
        pltpu.sync_copy(rows_v.at[i], acc_sh.at[dst_v.at[j]], add=True)
        if issue_next:
            pltpu.async_copy(y_hbm.at[src_v.at[pl.ds((j + NBUF) * K, K)]],
                             rows_v.at[i], sems[i])

    n_main = (NCHUNK - NBUF) // NBUF

    def body(g, carry):
        for i in range(NBUF):
            _step(g * NBUF + i, i, True)
        return carry

    lax.fori_loop(0, n_main, body, 0)
    for j in range(n_main * NBUF, NCHUNK):
        _step(j, j % NBUF, j + NBUF < NCHUNK)
    plsc.subcore_barrier()

    @pl.when(s < NS - 1)
    def _():
        pltpu.sync_copy(acc_sh.at[pl.ds(s * 632, 632)],
                        out_hbm.at[pl.ds(c * N + s * 632, 632)])

    @pl.when(s == NS - 1)
    def _():
        pltpu.sync_copy(acc_sh.at[pl.ds(15 * 632, 520)],
                        out_hbm.at[pl.ds(c * N + 15 * 632, 520)])


BR = 2000


def _tc_prescale_body(x_ref, w_ref, b_ref, d0_ref, d1_ref,
                      y_ref, r_ref, dis_ref):
    xw = jnp.dot(x_ref[...], w_ref[...], preferred_element_type=jnp.float32)
    degt = d0_ref[...] + d1_ref[...] + 1.0
    dis = lax.rsqrt(degt)
    y_ref[...] = xw * dis
    r_ref[...] = xw / degt + x_ref[...] + b_ref[...]
    dis_ref[...] = dis


def _tc_prescale(x, W, b2, d0, d1):
    grid = (N // BR,)
    return pl.pallas_call(
        _tc_prescale_body,
        grid=grid,
        in_specs=[
            pl.BlockSpec((BR, H), lambda i: (i, 0)),
            pl.BlockSpec((H, H), lambda i: (0, 0)),
            pl.BlockSpec((1, H), lambda i: (0, 0)),
            pl.BlockSpec((BR, 1), lambda i: (i, 0)),
            pl.BlockSpec((BR, 1), lambda i: (i, 0)),
        ],
        out_specs=[
            pl.BlockSpec((BR, H), lambda i: (i, 0)),
            pl.BlockSpec((BR, H), lambda i: (i, 0)),
            pl.BlockSpec((BR, 1), lambda i: (i, 0)),
        ],
        out_shape=[
            jax.ShapeDtypeStruct((N, H), jnp.float32),
            jax.ShapeDtypeStruct((N, H), jnp.float32),
            jax.ShapeDtypeStruct((N, 1), jnp.float32),
        ],
    )(x, W, b2, d0, d1)


def _tc_finish_body(p0_ref, p1_ref, r_ref, dis_ref, o_ref):
    h = dis_ref[...] * (p0_ref[...] + p1_ref[...]) + r_ref[...]
    mean = jnp.mean(h, axis=1, keepdims=True)
    cent = h - mean
    var = jnp.mean(cent * cent, axis=1, keepdims=True)
    o_ref[...] = cent * lax.rsqrt(var + 1e-5)


def _tc_finish(p0, p1, r, dis):
    grid = (N // BR,)
    return pl.pallas_call(
        _tc_finish_body,
        grid=grid,
        in_specs=[
            pl.BlockSpec((BR, H), lambda i: (i, 0)),
            pl.BlockSpec((BR, H), lambda i: (i, 0)),
            pl.BlockSpec((BR, H), lambda i: (i, 0)),
            pl.BlockSpec((BR, 1), lambda i: (i, 0)),
        ],
        out_specs=pl.BlockSpec((BR, H), lambda i: (i, 0)),
        out_shape=jax.ShapeDtypeStruct((N, H), jnp.float32),
    )(p0, p1, r, dis)


def kernel(x, edge_index, batch, W, b):
    src = edge_index[0].astype(jnp.int32)
    dst = edge_index[1].astype(jnp.int32)
    npad = EP - E
    pad_iota = jnp.arange(npad, dtype=jnp.int32)
    src_p = jnp.concatenate([src, pad_iota % N])
    dst_p = jnp.concatenate([dst, N + (pad_iota % 128)])
    src_p = src_p.reshape(NW, EWP)
    dst3 = dst_p.reshape(NW, NCHUNK, K)
    dstd = dst_p.reshape(NW, NCHUNK_D, KD)

    ones_k = jnp.ones((KD,), jnp.float32)
    zeros_2k = jnp.zeros((2000,), jnp.float32)
    zrows = jnp.zeros((648, H), jnp.float32)

    deg = _sc_degree(dstd, ones_k, zeros_2k)
    d0 = deg[:N].reshape(N, 1)
    d1 = deg[N:].reshape(N, 1)

    y, r, dis = _tc_prescale(x, W, b.reshape(1, H), d0, d1)

    acc = _sc_scatter(y, src_p, dst3, zrows)

    return _tc_finish(acc[:N], acc[N:], r, dis)


# speedup vs baseline: 2.4348x; 1.0407x over previous
import functools

import jax
import jax.numpy as jnp
from jax import lax
from jax.experimental import pallas as pl
from jax.experimental.pallas import tpu as pltpu
from jax.experimental.pallas import tpu_sc as plsc

N = 10000
H = 128
E = 320000
NC = 2
NS = 16
NW = NC * NS
EWP = 10240
EP = EWP * NW
K = 64
NCHUNK = EWP // K
KD = 128
NCHUNK_D = EWP // KD
NBUF = 2
ACC_R = N + 128
DEG_R = N + 2000

_mesh = plsc.VectorSubcoreMesh(
    core_axis_name="c", subcore_axis_name="s", num_cores=NC, num_subcores=NS)


@functools.partial(
    pl.kernel,
    out_type=jax.ShapeDtypeStruct((NC * N,), jnp.float32),
    mesh=_mesh,
    scratch_types=[
        pltpu.VMEM((NCHUNK_D, KD), jnp.int32),
        pltpu.VMEM((KD,), jnp.float32),
        pltpu.VMEM((2000,), jnp.float32),
        pltpu.VMEM_SHARED((DEG_R,), jnp.float32),
    ],
)
def _sc_degree(dst_hbm, ones_hbm, zeros_hbm, out_hbm,
               idx_v, ones_v, stage_v, deg_sh):
    c = lax.axis_index("c")
    s = lax.axis_index("s")
    wid = c * NS + s

    @pl.when(s == 0)
    def _():
        pltpu.sync_copy(zeros_hbm, stage_v)
        for t in range(DEG_R // 2000):
            pltpu.sync_copy(stage_v, deg_sh.at[pl.ds(t * 2000, 2000)])

    pltpu.sync_copy(dst_hbm.at[wid], idx_v)
    pltpu.sync_copy(ones_hbm, ones_v)
    plsc.subcore_barrier()

    def body(j, carry):
        pltpu.sync_copy(ones_v, deg_sh.at[idx_v.at[j]], add=True)
        return carry

    lax.fori_loop(0, NCHUNK_D, body, 0)
    plsc.subcore_barrier()

    @pl.when(s == 0)
    def _():
        for t in range(N // 2000):
            pltpu.sync_copy(deg_sh.at[pl.ds(t * 2000, 2000)], stage_v)
            pltpu.sync_copy(stage_v, out_hbm.at[pl.ds(c * N + t * 2000, 2000)])


@functools.partial(
    pl.kernel,
    out_type=jax.ShapeDtypeStruct((NC * N, H), jnp.float32),
    mesh=_mesh,
    scratch_types=[
        pltpu.VMEM((EWP,), jnp.int32),
        pltpu.VMEM((NCHUNK, K), jnp.int32),
        pltpu.VMEM((NBUF, K, H), jnp.float32),
        pltpu.VMEM_SHARED((ACC_R, H), jnp.float32),
        [pltpu.SemaphoreType.DMA] * NBUF,
    ],
)
def _sc_scatter(y_hbm, src_hbm, dst_hbm, zrows_hbm, out_hbm,
                src_v, dst_v, rows_v, acc_sh, sems):
    c = lax.axis_index("c")
    s = lax.axis_index("s")
    wid = c * NS + s

    @pl.when(s < NS - 1)
    def _():
        pltpu.sync_copy(zrows_hbm.at[pl.ds(0, 632)],
                        acc_sh.at[pl.ds(s * 632, 632)])

    @pl.when(s == NS - 1)
    def _():
        pltpu.sync_copy(zrows_hbm, acc_sh.at[pl.ds(15 * 632, 648)])

    pltpu.sync_copy(src_hbm.at[wid], src_v)
    pltpu.sync_copy(dst_hbm.at[wid], dst_v)
    plsc.subcore_barrier()

    for i in range(NBUF):
        pltpu.async_copy(y_hbm.at[src_v.at[pl.ds(i * K, K)]],
                         rows_v.at[i], sems[i])

    def _step(j, i, issue_next):
        pltpu.make_async_copy(y_hbm.at[src_v.at[pl.ds(j * K, K)]],
                              rows_v.at[i], sems[i]).wait()
        pltpu.sync_copy(rows_v.at[i], acc_sh.at[dst_v.at[j]], add=True)
        if issue_next:
            pltpu.async_copy(y_hbm.at[src_v.at[pl.ds((j + NBUF) * K, K)]],
                             rows_v.at[i], sems[i])

    n_main = (NCHUNK - NBUF) // NBUF

    def body(g, carry):
        for i in range(NBUF):
            _step(g * NBUF + i, i, True)
        return carry

    lax.fori_loop(0, n_main, body, 0)
    for j in range(n_main * NBUF, NCHUNK):
        _step(j, j % NBUF, j + NBUF < NCHUNK)
    plsc.subcore_barrier()

    @pl.when(s < NS - 1)
    def _():
        pltpu.sync_copy(acc_sh.at[pl.ds(s * 632, 632)],
                        out_hbm.at[pl.ds(c * N + s * 632, 632)])

    @pl.when(s == NS - 1)
    def _():
        pltpu.sync_copy(acc_sh.at[pl.ds(15 * 632, 520)],
                        out_hbm.at[pl.ds(c * N + 15 * 632, 520)])


BR = 2000


def _tc_prescale_body(x_ref, w_ref, b_ref, d0_ref, d1_ref,
                      y_ref, r_ref, dis_ref):
    xw = jnp.dot(x_ref[...], w_ref[...], preferred_element_type=jnp.float32)
    degt = d0_ref[...] + d1_ref[...] + 1.0
    dis = lax.rsqrt(degt)
    y_ref[...] = xw * dis
    r_ref[...] = xw / degt + x_ref[...] + b_ref[...]
    dis_ref[...] = dis


def _tc_prescale(x, W, b2, deg2):
    grid = (N // BR,)
    return pl.pallas_call(
        _tc_prescale_body,
        grid=grid,
        in_specs=[
            pl.BlockSpec((BR, H), lambda i: (i, 0)),
            pl.BlockSpec((H, H), lambda i: (0, 0)),
            pl.BlockSpec((1, H), lambda i: (0, 0)),
            pl.BlockSpec((BR, 1), lambda i: (i, 0)),
            pl.BlockSpec((BR, 1), lambda i: (N // BR + i, 0)),
        ],
        out_specs=[
            pl.BlockSpec((BR, H), lambda i: (i, 0)),
            pl.BlockSpec((BR, H), lambda i: (i, 0)),
            pl.BlockSpec((BR, 1), lambda i: (i, 0)),
        ],
        out_shape=[
            jax.ShapeDtypeStruct((N, H), jnp.float32),
            jax.ShapeDtypeStruct((N, H), jnp.float32),
            jax.ShapeDtypeStruct((N, 1), jnp.float32),
        ],
    )(x, W, b2, deg2, deg2)


def _tc_finish_body(p0_ref, p1_ref, r_ref, dis_ref, o_ref):
    h = dis_ref[...] * (p0_ref[...] + p1_ref[...]) + r_ref[...]
    mean = jnp.mean(h, axis=1, keepdims=True)
    cent = h - mean
    var = jnp.mean(cent * cent, axis=1, keepdims=True)
    o_ref[...] = cent * lax.rsqrt(var + 1e-5)


def _tc_finish(acc, r, dis):
    grid = (N // BR,)
    return pl.pallas_call(
        _tc_finish_body,
        grid=grid,
        in_specs=[
            pl.BlockSpec((BR, H), lambda i: (i, 0)),
            pl.BlockSpec((BR, H), lambda i: (N // BR + i, 0)),
            pl.BlockSpec((BR, H), lambda i: (i, 0)),
            pl.BlockSpec((BR, 1), lambda i: (i, 0)),
        ],
        out_specs=pl.BlockSpec((BR, H), lambda i: (i, 0)),
        out_shape=jax.ShapeDtypeStruct((N, H), jnp.float32),
    )(acc, acc, r, dis)


def kernel(x, edge_index, batch, W, b):
    src = edge_index[0].astype(jnp.int32)
    dst = edge_index[1].astype(jnp.int32)
    npad = EP - E
    pad_iota = jnp.arange(npad, dtype=jnp.int32)
    src_p = jnp.concatenate([src, pad_iota % N])
    dst_p = jnp.concatenate([dst, N + (pad_iota % 128)])
    src_p = src_p.reshape(NW, EWP)
    dst3 = dst_p.reshape(NW, NCHUNK, K)
    dstd = dst_p.reshape(NW, NCHUNK_D, KD)

    ones_k = jnp.ones((KD,), jnp.float32)
    zeros_2k = jnp.zeros((2000,), jnp.float32)
    zrows = jnp.zeros((648, H), jnp.float32)

    deg = _sc_degree(dstd, ones_k, zeros_2k)

    y, r, dis = _tc_prescale(x, W, b.reshape(1, H), deg.reshape(NC * N, 1))

    acc = _sc_scatter(y, src_p, dst3, zrows)

    return _tc_finish(acc, r, dis)


# speedup vs baseline: 2.6897x; 1.1047x over previous
import functools

import jax
import jax.numpy as jnp
from jax import lax
from jax.experimental import pallas as pl
from jax.experimental.pallas import tpu as pltpu
from jax.experimental.pallas import tpu_sc as plsc

N = 10000
H = 128
E = 320000
NC = 2
NS = 16
NW = NC * NS
EWP = 10240
EP = EWP * NW
K = 128
NCHUNK = EWP // K
PH = 2
CPP = NCHUNK // PH
IDXW = CPP * K
KD = 128
NCHUNK_D = EWP // KD
NBUF = 2
ACC_R = N + 8
DEG_R = N + 2000

_mesh = plsc.VectorSubcoreMesh(
    core_axis_name="c", subcore_axis_name="s", num_cores=NC, num_subcores=NS)


@functools.partial(
    pl.kernel,
    out_type=jax.ShapeDtypeStruct((NC * N,), jnp.float32),
    mesh=_mesh,
    scratch_types=[
        pltpu.VMEM((NCHUNK_D, KD), jnp.int32),
        pltpu.VMEM((KD,), jnp.float32),
        pltpu.VMEM((2000,), jnp.float32),
        pltpu.VMEM_SHARED((DEG_R,), jnp.float32),
    ],
)
def _sc_degree(dst_hbm, ones_hbm, zeros_hbm, out_hbm,
               idx_v, ones_v, stage_v, deg_sh):
    c = lax.axis_index("c")
    s = lax.axis_index("s")
    wid = c * NS + s

    @pl.when(s == 0)
    def _():
        pltpu.sync_copy(zeros_hbm, stage_v)
        for t in range(DEG_R // 2000):
            pltpu.sync_copy(stage_v, deg_sh.at[pl.ds(t * 2000, 2000)])

    pltpu.sync_copy(dst_hbm.at[wid], idx_v)
    pltpu.sync_copy(ones_hbm, ones_v)
    plsc.subcore_barrier()

    def body(j, carry):
        pltpu.sync_copy(ones_v, deg_sh.at[idx_v.at[j]], add=True)
        return carry

    lax.fori_loop(0, NCHUNK_D, body, 0)
    plsc.subcore_barrier()

    @pl.when(s == 0)
    def _():
        for t in range(N // 2000):
            pltpu.sync_copy(deg_sh.at[pl.ds(t * 2000, 2000)], stage_v)
            pltpu.sync_copy(stage_v, out_hbm.at[pl.ds(c * N + t * 2000, 2000)])


@functools.partial(
    pl.kernel,
    out_type=jax.ShapeDtypeStruct((NC * N, H), jnp.float32),
    mesh=_mesh,
    scratch_types=[
        pltpu.VMEM((IDXW,), jnp.int32),
        pltpu.VMEM((IDXW,), jnp.int32),
        pltpu.VMEM((NBUF, K, H), jnp.float32),
        pltpu.VMEM_SHARED((ACC_R, H), jnp.float32),
        [pltpu.SemaphoreType.DMA] * NBUF,
    ],
)
def _sc_scatter(y_hbm, src_hbm, dst_hbm, zrows_hbm, out_hbm,
                src_v, dst_v, rows_v, acc_sh, gsems):
    c = lax.axis_index("c")
    s = lax.axis_index("s")
    wid = c * NS + s

    @pl.when(s < NS - 1)
    def _():
        pltpu.sync_copy(zrows_hbm, acc_sh.at[pl.ds(s * 632, 632)])

    @pl.when(s == NS - 1)
    def _():
        pltpu.sync_copy(zrows_hbm.at[pl.ds(0, 528)],
                        acc_sh.at[pl.ds(15 * 632, 528)])

    plsc.subcore_barrier()

    def _gather(m, slot):
        pltpu.async_copy(y_hbm.at[src_v.at[pl.ds(m * K, K)]],
                         rows_v.at[slot], gsems[slot])

    def _gwait(m, slot):
        pltpu.make_async_copy(y_hbm.at[src_v.at[pl.ds(m * K, K)]],
                              rows_v.at[slot], gsems[slot]).wait()

    def _step(j, i, issue_next):
        _gwait(j, i)
        pltpu.sync_copy(rows_v.at[i],
                        acc_sh.at[dst_v.at[pl.ds(j * K, K)]], add=True)
        if issue_next:
            _gather(j + NBUF, i)

    for p in range(PH):
        pltpu.sync_copy(src_hbm.at[wid * PH + p], src_v)
        pltpu.sync_copy(dst_hbm.at[wid * PH + p], dst_v)
        for i in range(NBUF):
            _gather(i, i)

        def body(g, carry):
            for i in range(NBUF):
                _step(g * NBUF + i, i, True)
            return carry

        lax.fori_loop(0, (CPP - NBUF) // NBUF, body, 0)
        for j in range(CPP - NBUF, CPP):
            _step(j, j % NBUF, False)
    plsc.subcore_barrier()

    @pl.when(s < NS - 1)
    def _():
        pltpu.sync_copy(acc_sh.at[pl.ds(s * 632, 632)],
                        out_hbm.at[pl.ds(c * N + s * 632, 632)])

    @pl.when(s == NS - 1)
    def _():
        pltpu.sync_copy(acc_sh.at[pl.ds(15 * 632, 520)],
                        out_hbm.at[pl.ds(c * N + 15 * 632, 520)])


BR = 2000


def _tc_prescale_body(x_ref, w_ref, b_ref, d0_ref, d1_ref,
                      y_ref, r_ref, dis_ref):
    xw = jnp.dot(x_ref[...], w_ref[...], preferred_element_type=jnp.float32)
    degt = d0_ref[...] + d1_ref[...] + 1.0
    dis = lax.rsqrt(degt)
    y_ref[...] = xw * dis
    r_ref[...] = xw / degt + x_ref[...] + b_ref[...]
    dis_ref[...] = dis


def _tc_prescale(x, W, b2, deg2):
    grid = (N // BR,)
    return pl.pallas_call(
        _tc_prescale_body,
        grid=grid,
        in_specs=[
            pl.BlockSpec((BR, H), lambda i: (i, 0)),
            pl.BlockSpec((H, H), lambda i: (0, 0)),
            pl.BlockSpec((1, H), lambda i: (0, 0)),
            pl.BlockSpec((BR, 1), lambda i: (i, 0)),
            pl.BlockSpec((BR, 1), lambda i: (N // BR + i, 0)),
        ],
        out_specs=[
            pl.BlockSpec((BR, H), lambda i: (i, 0)),
            pl.BlockSpec((BR, H), lambda i: (i, 0)),
            pl.BlockSpec((BR, 1), lambda i: (i, 0)),
        ],
        out_shape=[
            jax.ShapeDtypeStruct((N, H), jnp.float32),
            jax.ShapeDtypeStruct((N, H), jnp.float32),
            jax.ShapeDtypeStruct((N, 1), jnp.float32),
        ],
    )(x, W, b2, deg2, deg2)


def _tc_finish_body(p0_ref, p1_ref, r_ref, dis_ref, o_ref):
    h = dis_ref[...] * (p0_ref[...] + p1_ref[...]) + r_ref[...]
    mean = jnp.mean(h, axis=1, keepdims=True)
    cent = h - mean
    var = jnp.mean(cent * cent, axis=1, keepdims=True)
    o_ref[...] = cent * lax.rsqrt(var + 1e-5)


def _tc_finish(acc, r, dis):
    grid = (N // BR,)
    return pl.pallas_call(
        _tc_finish_body,
        grid=grid,
        in_specs=[
            pl.BlockSpec((BR, H), lambda i: (i, 0)),
            pl.BlockSpec((BR, H), lambda i: (N // BR + i, 0)),
            pl.BlockSpec((BR, H), lambda i: (i, 0)),
            pl.BlockSpec((BR, 1), lambda i: (i, 0)),
        ],
        out_specs=pl.BlockSpec((BR, H), lambda i: (i, 0)),
        out_shape=jax.ShapeDtypeStruct((N, H), jnp.float32),
    )(acc, acc, r, dis)


def kernel(x, edge_index, batch, W, b):
    src = edge_index[0].astype(jnp.int32)
    dst = edge_index[1].astype(jnp.int32)
    nduw = EWP - E // NW
    pad_iota = jnp.arange(nduw, dtype=jnp.int32)
    dsrc = jnp.broadcast_to((pad_iota * 37) % N, (NW, nduw))
    ddst = jnp.broadcast_to(N + (pad_iota % 8), (NW, nduw))
    src_p = jnp.concatenate([src.reshape(NW, E // NW), dsrc], axis=1)
    dst_p = jnp.concatenate([dst.reshape(NW, E // NW), ddst], axis=1)
    src_p = src_p.reshape(NW * PH, IDXW)
    dst2 = dst_p.reshape(NW * PH, IDXW)
    dstd = dst_p.reshape(NW, NCHUNK_D, KD)

    ones_k = jnp.ones((KD,), jnp.float32)
    zeros_2k = jnp.zeros((2000,), jnp.float32)
    zrows = jnp.zeros((632, H), jnp.float32)

    deg = _sc_degree(dstd, ones_k, zeros_2k)

    y, r, dis = _tc_prescale(x, W, b.reshape(1, H), deg.reshape(NC * N, 1))

    acc = _sc_scatter(y, src_p, dst2, zrows)

    return _tc_finish(acc, r, dis)


# speedup vs baseline: 2.7179x; 1.0105x over previous
import functools

import jax
import jax.numpy as jnp
from jax import lax
from jax.experimental import pallas as pl
from jax.experimental.pallas import tpu as pltpu
from jax.experimental.pallas import tpu_sc as plsc

N = 10000
H = 128
E = 320000
NC = 2
NS = 16
NW = NC * NS
EW = E // NW
K = 128
PH = 2
PHW = EW // PH
FULL = PHW // K
TAIL = PHW - FULL * K
KD = 128
DFULL = EW // KD
DTAIL = EW - DFULL * KD
NBUF = 2
DEG_R = N + 2000

_mesh = plsc.VectorSubcoreMesh(
    core_axis_name="c", subcore_axis_name="s", num_cores=NC, num_subcores=NS)


@functools.partial(
    pl.kernel,
    out_type=jax.ShapeDtypeStruct((NC * N,), jnp.float32),
    mesh=_mesh,
    scratch_types=[
        pltpu.VMEM((EW,), jnp.int32),
        pltpu.VMEM((KD,), jnp.float32),
        pltpu.VMEM((2000,), jnp.float32),
        pltpu.VMEM_SHARED((DEG_R,), jnp.float32),
    ],
)
def _sc_degree(dst_hbm, ones_hbm, zeros_hbm, out_hbm,
               idx_v, ones_v, stage_v, deg_sh):
    c = lax.axis_index("c")
    s = lax.axis_index("s")
    wid = c * NS + s

    @pl.when(s == 0)
    def _():
        pltpu.sync_copy(zeros_hbm, stage_v)
        for t in range(DEG_R // 2000):
            pltpu.sync_copy(stage_v, deg_sh.at[pl.ds(t * 2000, 2000)])

    pltpu.sync_copy(dst_hbm.at[wid], idx_v)
    pltpu.sync_copy(ones_hbm, ones_v)
    plsc.subcore_barrier()

    def body(j, carry):
        pltpu.sync_copy(ones_v, deg_sh.at[idx_v.at[pl.ds(j * KD, KD)]],
                        add=True)
        return carry

    lax.fori_loop(0, DFULL, body, 0)
    pltpu.sync_copy(ones_v.at[pl.ds(0, DTAIL)],
                    deg_sh.at[idx_v.at[pl.ds(DFULL * KD, DTAIL)]], add=True)
    plsc.subcore_barrier()

    @pl.when(s == 0)
    def _():
        for t in range(N // 2000):
            pltpu.sync_copy(deg_sh.at[pl.ds(t * 2000, 2000)], stage_v)
            pltpu.sync_copy(stage_v, out_hbm.at[pl.ds(c * N + t * 2000, 2000)])


@functools.partial(
    pl.kernel,
    out_type=jax.ShapeDtypeStruct((NC * N, H), jnp.float32),
    mesh=_mesh,
    scratch_types=[
        pltpu.VMEM((PHW,), jnp.int32),
        pltpu.VMEM((PHW,), jnp.int32),
        pltpu.VMEM((NBUF, K, H), jnp.float32),
        pltpu.VMEM_SHARED((N, H), jnp.float32),
        [pltpu.SemaphoreType.DMA] * NBUF,
    ],
)
def _sc_scatter(y_hbm, src_hbm, dst_hbm, zrows_hbm, out_hbm,
                src_v, dst_v, rows_v, acc_sh, gsems):
    c = lax.axis_index("c")
    s = lax.axis_index("s")
    wid = c * NS + s

    @pl.when(s < NS - 1)
    def _():
        pltpu.sync_copy(zrows_hbm, acc_sh.at[pl.ds(s * 632, 632)])

    @pl.when(s == NS - 1)
    def _():
        pltpu.sync_copy(zrows_hbm.at[pl.ds(0, 520)],
                        acc_sh.at[pl.ds(15 * 632, 520)])

    plsc.subcore_barrier()

    def _gather(m, slot):
        pltpu.async_copy(y_hbm.at[src_v.at[pl.ds(m * K, K)]],
                         rows_v.at[slot], gsems[slot])

    def _gwait(m, slot):
        pltpu.make_async_copy(y_hbm.at[src_v.at[pl.ds(m * K, K)]],
                              rows_v.at[slot], gsems[slot]).wait()

    def _step(j, i, issue_next):
        _gwait(j, i)
        pltpu.sync_copy(rows_v.at[i],
                        acc_sh.at[dst_v.at[pl.ds(j * K, K)]], add=True)
        if issue_next:
            _gather(j + NBUF, i)

    for p in range(PH):
        pltpu.sync_copy(src_hbm.at[wid * PH + p], src_v)
        pltpu.sync_copy(dst_hbm.at[wid * PH + p], dst_v)
        for i in range(NBUF):
            _gather(i, i)

        def body(g, carry):
            for i in range(NBUF):
                _step(g * NBUF + i, i, True)
            return carry

        n_main = (FULL - 3) // NBUF
        lax.fori_loop(0, n_main, body, 0)
        for j in range(n_main * NBUF, FULL):
            _step(j, j % NBUF, j + NBUF < FULL)
        pltpu.async_copy(y_hbm.at[src_v.at[pl.ds(FULL * K, TAIL)]],
                         rows_v.at[0, pl.ds(0, TAIL)], gsems[0]).wait()
        pltpu.sync_copy(rows_v.at[0, pl.ds(0, TAIL)],
                        acc_sh.at[dst_v.at[pl.ds(FULL * K, TAIL)]], add=True)
    plsc.subcore_barrier()

    @pl.when(s < NS - 1)
    def _():
        pltpu.sync_copy(acc_sh.at[pl.ds(s * 632, 632)],
                        out_hbm.at[pl.ds(c * N + s * 632, 632)])

    @pl.when(s == NS - 1)
    def _():
        pltpu.sync_copy(acc_sh.at[pl.ds(15 * 632, 520)],
                        out_hbm.at[pl.ds(c * N + 15 * 632, 520)])


BR = 2000


def _tc_prescale_body(x_ref, w_ref, b_ref, d0_ref, d1_ref,
                      y_ref, r_ref, dis_ref):
    xw = jnp.dot(x_ref[...], w_ref[...], preferred_element_type=jnp.float32)
    degt = d0_ref[...] + d1_ref[...] + 1.0
    dis = lax.rsqrt(degt)
    y_ref[...] = xw * dis
    r_ref[...] = xw / degt + x_ref[...] + b_ref[...]
    dis_ref[...] = dis


def _tc_prescale(x, W, b2, deg2):
    grid = (N // BR,)
    return pl.pallas_call(
        _tc_prescale_body,
        grid=grid,
        in_specs=[
            pl.BlockSpec((BR, H), lambda i: (i, 0)),
            pl.BlockSpec((H, H), lambda i: (0, 0)),
            pl.BlockSpec((1, H), lambda i: (0, 0)),
            pl.BlockSpec((BR, 1), lambda i: (i, 0)),
            pl.BlockSpec((BR, 1), lambda i: (N // BR + i, 0)),
        ],
        out_specs=[
            pl.BlockSpec((BR, H), lambda i: (i, 0)),
            pl.BlockSpec((BR, H), lambda i: (i, 0)),
            pl.BlockSpec((BR, 1), lambda i: (i, 0)),
        ],
        out_shape=[
            jax.ShapeDtypeStruct((N, H), jnp.float32),
            jax.ShapeDtypeStruct((N, H), jnp.float32),
            jax.ShapeDtypeStruct((N, 1), jnp.float32),
        ],
    )(x, W, b2, deg2, deg2)


def _tc_finish_body(p0_ref, p1_ref, r_ref, dis_ref, o_ref):
    h = dis_ref[...] * (p0_ref[...] + p1_ref[...]) + r_ref[...]
    mean = jnp.mean(h, axis=1, keepdims=True)
    cent = h - mean
    var = jnp.mean(cent * cent, axis=1, keepdims=True)
    o_ref[...] = cent * lax.rsqrt(var + 1e-5)


def _tc_finish(acc, r, dis):
    grid = (N // BR,)
    return pl.pallas_call(
        _tc_finish_body,
        grid=grid,
        in_specs=[
            pl.BlockSpec((BR, H), lambda i: (i, 0)),
            pl.BlockSpec((BR, H), lambda i: (N // BR + i, 0)),
            pl.BlockSpec((BR, H), lambda i: (i, 0)),
            pl.BlockSpec((BR, 1), lambda i: (i, 0)),
        ],
        out_specs=pl.BlockSpec((BR, H), lambda i: (i, 0)),
        out_shape=jax.ShapeDtypeStruct((N, H), jnp.float32),
    )(acc, acc, r, dis)


def kernel(x, edge_index, batch, W, b):
    src = edge_index[0].astype(jnp.int32)
    dst = edge_index[1].astype(jnp.int32)
    src_p = src.reshape(NW * PH, PHW)
    dst2 = dst.reshape(NW * PH, PHW)
    dstd = dst.reshape(NW, EW)

    ones_k = jnp.ones((KD,), jnp.float32)
    zeros_2k = jnp.zeros((2000,), jnp.float32)
    zrows = jnp.zeros((632, H), jnp.float32)

    deg = _sc_degree(dstd, ones_k, zeros_2k)

    y, r, dis = _tc_prescale(x, W, b.reshape(1, H), deg.reshape(NC * N, 1))

    acc = _sc_scatter(y, src_p, dst2, zrows)

    return _tc_finish(acc, r, dis)


# speedup vs baseline: 2.8251x; 1.0395x over previous
import functools

import jax
import jax.numpy as jnp
from jax import lax
from jax.experimental import pallas as pl
from jax.experimental.pallas import tpu as pltpu
from jax.experimental.pallas import tpu_sc as plsc

N = 10000
H = 128
E = 320000
NC = 2
NS = 16
NW = NC * NS
EW = E // NW
K = 128
PH = 2
PHW = EW // PH
FULL = PHW // K
TAIL = PHW - FULL * K
KD = 128
DFULL = EW // KD
DTAIL = EW - DFULL * KD
NBUF = 2
DEG_R = N + 2000

_mesh = plsc.VectorSubcoreMesh(
    core_axis_name="c", subcore_axis_name="s", num_cores=NC, num_subcores=NS)


@functools.partial(
    pl.kernel,
    out_type=jax.ShapeDtypeStruct((NC * N,), jnp.float32),
    mesh=_mesh,
    scratch_types=[
        pltpu.VMEM((EW,), jnp.int32),
        pltpu.VMEM((KD,), jnp.float32),
        pltpu.VMEM((2000,), jnp.float32),
        pltpu.VMEM_SHARED((DEG_R,), jnp.float32),
    ],
)
def _sc_degree(ei_hbm, ones_hbm, zeros_hbm, out_hbm,
               idx_v, ones_v, stage_v, deg_sh):
    c = lax.axis_index("c")
    s = lax.axis_index("s")
    wid = c * NS + s

    @pl.when(s == 0)
    def _():
        pltpu.sync_copy(zeros_hbm, stage_v)
        for t in range(DEG_R // 2000):
            pltpu.sync_copy(stage_v, deg_sh.at[pl.ds(t * 2000, 2000)])

    pltpu.sync_copy(ei_hbm.at[pl.ds(E + wid * EW, EW)], idx_v)
    pltpu.sync_copy(ones_hbm, ones_v)
    plsc.subcore_barrier()

    def body(j, carry):
        pltpu.sync_copy(ones_v, deg_sh.at[idx_v.at[pl.ds(j * KD, KD)]],
                        add=True)
        return carry

    lax.fori_loop(0, DFULL, body, 0)
    pltpu.sync_copy(ones_v.at[pl.ds(0, DTAIL)],
                    deg_sh.at[idx_v.at[pl.ds(DFULL * KD, DTAIL)]], add=True)
    plsc.subcore_barrier()

    @pl.when(s == 0)
    def _():
        for t in range(N // 2000):
            pltpu.sync_copy(deg_sh.at[pl.ds(t * 2000, 2000)], stage_v)
            pltpu.sync_copy(stage_v, out_hbm.at[pl.ds(c * N + t * 2000, 2000)])


@functools.partial(
    pl.kernel,
    out_type=jax.ShapeDtypeStruct((NC * N, H), jnp.float32),
    mesh=_mesh,
    scratch_types=[
        pltpu.VMEM((PHW,), jnp.int32),
        pltpu.VMEM((PHW,), jnp.int32),
        pltpu.VMEM((NBUF, K, H), jnp.float32),
        pltpu.VMEM_SHARED((N, H), jnp.float32),
        [pltpu.SemaphoreType.DMA] * NBUF,
    ],
)
def _sc_scatter(y_hbm, ei_hbm, zrows_hbm, out_hbm,
                src_v, dst_v, rows_v, acc_sh, gsems):
    c = lax.axis_index("c")
    s = lax.axis_index("s")
    wid = c * NS + s

    @pl.when(s < NS - 1)
    def _():
        pltpu.sync_copy(zrows_hbm, acc_sh.at[pl.ds(s * 632, 632)])

    @pl.when(s == NS - 1)
    def _():
        pltpu.sync_copy(zrows_hbm.at[pl.ds(0, 520)],
                        acc_sh.at[pl.ds(15 * 632, 520)])

    plsc.subcore_barrier()

    def _gather(m, slot):
        pltpu.async_copy(y_hbm.at[src_v.at[pl.ds(m * K, K)]],
                         rows_v.at[slot], gsems[slot])

    def _gwait(m, slot):
        pltpu.make_async_copy(y_hbm.at[src_v.at[pl.ds(m * K, K)]],
                              rows_v.at[slot], gsems[slot]).wait()

    def _step(j, i, issue_next):
        _gwait(j, i)
        pltpu.sync_copy(rows_v.at[i],
                        acc_sh.at[dst_v.at[pl.ds(j * K, K)]], add=True)
        if issue_next:
            _gather(j + NBUF, i)

    for p in range(PH):
        base = (wid * PH + p) * PHW
        pltpu.sync_copy(ei_hbm.at[pl.ds(base, PHW)], src_v)
        pltpu.sync_copy(ei_hbm.at[pl.ds(E + base, PHW)], dst_v)
        for i in range(NBUF):
            _gather(i, i)

        def body(g, carry):
            for i in range(NBUF):
                _step(g * NBUF + i, i, True)
            return carry

        n_main = (FULL - 3) // NBUF
        lax.fori_loop(0, n_main, body, 0)
        for j in range(n_main * NBUF, FULL):
            _step(j, j % NBUF, j + NBUF < FULL)
        pltpu.async_copy(y_hbm.at[src_v.at[pl.ds(FULL * K, TAIL)]],
                         rows_v.at[0, pl.ds(0, TAIL)], gsems[0]).wait()
        pltpu.sync_copy(rows_v.at[0, pl.ds(0, TAIL)],
                        acc_sh.at[dst_v.at[pl.ds(FULL * K, TAIL)]], add=True)
    plsc.subcore_barrier()

    @pl.when(s < NS - 1)
    def _():
        pltpu.sync_copy(acc_sh.at[pl.ds(s * 632, 632)],
                        out_hbm.at[pl.ds(c * N + s * 632, 632)])

    @pl.when(s == NS - 1)
    def _():
        pltpu.sync_copy(acc_sh.at[pl.ds(15 * 632, 520)],
                        out_hbm.at[pl.ds(c * N + 15 * 632, 520)])


BR = 1000


def _tc_prescale_body(x_ref, w_ref, b_ref, d0_ref, d1_ref,
                      y_ref, r_ref, dis_ref):
    xw = jnp.dot(x_ref[...], w_ref[...], preferred_element_type=jnp.float32)
    degt = d0_ref[...] + d1_ref[...] + 1.0
    dis = lax.rsqrt(degt)
    y_ref[...] = xw * dis
    r_ref[...] = xw / degt + x_ref[...] + b_ref[...]
    dis_ref[...] = dis


def _tc_prescale(x, W, b2, deg2):
    grid = (N // BR,)
    return pl.pallas_call(
        _tc_prescale_body,
        grid=grid,
        in_specs=[
            pl.BlockSpec((BR, H), lambda i: (i, 0)),
            pl.BlockSpec((H, H), lambda i: (0, 0)),
            pl.BlockSpec((1, H), lambda i: (0, 0)),
            pl.BlockSpec((BR, 1), lambda i: (i, 0)),
            pl.BlockSpec((BR, 1), lambda i: (N // BR + i, 0)),
        ],
        out_specs=[
            pl.BlockSpec((BR, H), lambda i: (i, 0)),
            pl.BlockSpec((BR, H), lambda i: (i, 0)),
            pl.BlockSpec((BR, 1), lambda i: (i, 0)),
        ],
        out_shape=[
            jax.ShapeDtypeStruct((N, H), jnp.float32),
            jax.ShapeDtypeStruct((N, H), jnp.float32),
            jax.ShapeDtypeStruct((N, 1), jnp.float32),
        ],
    )(x, W, b2, deg2, deg2)


def _tc_finish_body(p0_ref, p1_ref, r_ref, dis_ref, o_ref):
    h = dis_ref[...] * (p0_ref[...] + p1_ref[...]) + r_ref[...]
    mean = jnp.mean(h, axis=1, keepdims=True)
    cent = h - mean
    var = jnp.mean(cent * cent, axis=1, keepdims=True)
    o_ref[...] = cent * lax.rsqrt(var + 1e-5)


def _tc_finish(acc, r, dis):
    grid = (N // BR,)
    return pl.pallas_call(
        _tc_finish_body,
        grid=grid,
        in_specs=[
            pl.BlockSpec((BR, H), lambda i: (i, 0)),
            pl.BlockSpec((BR, H), lambda i: (N // BR + i, 0)),
            pl.BlockSpec((BR, H), lambda i: (i, 0)),
            pl.BlockSpec((BR, 1), lambda i: (i, 0)),
        ],
        out_specs=pl.BlockSpec((BR, H), lambda i: (i, 0)),
        out_shape=jax.ShapeDtypeStruct((N, H), jnp.float32),
    )(acc, acc, r, dis)


def kernel(x, edge_index, batch, W, b):
    ei = edge_index.astype(jnp.int32).reshape(2 * E)

    ones_k = jnp.ones((KD,), jnp.float32)
    zeros_2k = jnp.zeros((2000,), jnp.float32)
    zrows = jnp.zeros((632, H), jnp.float32)

    deg = _sc_degree(ei, ones_k, zeros_2k)

    y, r, dis = _tc_prescale(x, W, b.reshape(1, H), deg.reshape(NC * N, 1))

    acc = _sc_scatter(y, ei, zrows)

    return _tc_finish(acc, r, dis)


# speedup vs baseline: 2.8971x; 1.0255x over previous
import functools

import jax
import jax.numpy as jnp
from jax import lax
from jax.experimental import pallas as pl
from jax.experimental.pallas import tpu as pltpu
from jax.experimental.pallas import tpu_sc as plsc

N = 10000
H = 128
E = 320000
NC = 2
NS = 16
NW = NC * NS
EW = E // NW
K = 128
PH = 2
PHW = EW // PH
FULL = PHW // K
TAIL = PHW - FULL * K
KD = 128
DFULL = EW // KD
DTAIL = EW - DFULL * KD
NBUF = 2
DEG_R = N + 2000

_mesh = plsc.VectorSubcoreMesh(
    core_axis_name="c", subcore_axis_name="s", num_cores=NC, num_subcores=NS)


@functools.partial(
    pl.kernel,
    out_type=jax.ShapeDtypeStruct((NC * N,), jnp.float32),
    mesh=_mesh,
    scratch_types=[
        pltpu.VMEM((EW,), jnp.int32),
        pltpu.VMEM((KD,), jnp.float32),
        pltpu.VMEM((2000,), jnp.float32),
        pltpu.VMEM_SHARED((DEG_R,), jnp.float32),
    ],
)
def _sc_degree(ei_hbm, ones_hbm, zeros_hbm, out_hbm,
               idx_v, ones_v, stage_v, deg_sh):
    c = lax.axis_index("c")
    s = lax.axis_index("s")
    wid = c * NS + s

    @pl.when(s == 0)
    def _():
        pltpu.sync_copy(zeros_hbm, stage_v)
        for t in range(DEG_R // 2000):
            pltpu.sync_copy(stage_v, deg_sh.at[pl.ds(t * 2000, 2000)])

    pltpu.sync_copy(ei_hbm.at[pl.ds(E + wid * EW, EW)], idx_v)
    pltpu.sync_copy(ones_hbm, ones_v)
    plsc.subcore_barrier()

    def body(j, carry):
        pltpu.sync_copy(ones_v, deg_sh.at[idx_v.at[pl.ds(j * KD, KD)]],
                        add=True)
        return carry

    lax.fori_loop(0, DFULL, body, 0)
    pltpu.sync_copy(ones_v.at[pl.ds(0, DTAIL)],
                    deg_sh.at[idx_v.at[pl.ds(DFULL * KD, DTAIL)]], add=True)
    plsc.subcore_barrier()

    @pl.when(s == 0)
    def _():
        for t in range(N // 2000):
            pltpu.sync_copy(deg_sh.at[pl.ds(t * 2000, 2000)], stage_v)
            pltpu.sync_copy(stage_v, out_hbm.at[pl.ds(c * N + t * 2000, 2000)])


@functools.partial(
    pl.kernel,
    out_type=jax.ShapeDtypeStruct((NC * N, H), jnp.float32),
    mesh=_mesh,
    scratch_types=[
        pltpu.VMEM((PHW,), jnp.int32),
        pltpu.VMEM((PHW,), jnp.int32),
        pltpu.VMEM((NBUF, K, H), jnp.float32),
        pltpu.VMEM_SHARED((N, H), jnp.float32),
        [pltpu.SemaphoreType.DMA] * NBUF,
    ],
)
def _sc_scatter(y_hbm, ei_hbm, zrows_hbm, out_hbm,
                src_v, dst_v, rows_v, acc_sh, gsems):
    c = lax.axis_index("c")
    s = lax.axis_index("s")
    wid = c * NS + s

    @pl.when(s < NS - 1)
    def _():
        pltpu.sync_copy(zrows_hbm, acc_sh.at[pl.ds(s * 632, 632)])

    @pl.when(s == NS - 1)
    def _():
        pltpu.sync_copy(zrows_hbm.at[pl.ds(0, 520)],
                        acc_sh.at[pl.ds(15 * 632, 520)])

    plsc.subcore_barrier()

    def _gather(m, slot):
        pltpu.async_copy(y_hbm.at[src_v.at[pl.ds(m * K, K)]],
                         rows_v.at[slot], gsems[slot])

    def _gwait(m, slot):
        pltpu.make_async_copy(y_hbm.at[src_v.at[pl.ds(m * K, K)]],
                              rows_v.at[slot], gsems[slot]).wait()

    def _step(j, i, issue_next):
        _gwait(j, i)
        pltpu.sync_copy(rows_v.at[i],
                        acc_sh.at[dst_v.at[pl.ds(j * K, K)]], add=True)
        if issue_next:
            _gather(j + NBUF, i)

    for p in range(PH):
        base = (wid * PH + p) * PHW
        pltpu.sync_copy(ei_hbm.at[pl.ds(base, PHW)], src_v)
        pltpu.sync_copy(ei_hbm.at[pl.ds(E + base, PHW)], dst_v)
        for i in range(NBUF):
            _gather(i, i)

        def body(g, carry):
            for i in range(NBUF):
                _step(g * NBUF + i, i, True)
            return carry

        n_main = (FULL - 3) // NBUF
        lax.fori_loop(0, n_main, body, 0)
        for j in range(n_main * NBUF, FULL):
            _step(j, j % NBUF, j + NBUF < FULL)
        pltpu.async_copy(y_hbm.at[src_v.at[pl.ds(FULL * K, TAIL)]],
                         rows_v.at[0, pl.ds(0, TAIL)], gsems[0]).wait()
        pltpu.sync_copy(rows_v.at[0, pl.ds(0, TAIL)],
                        acc_sh.at[dst_v.at[pl.ds(FULL * K, TAIL)]], add=True)
    plsc.subcore_barrier()

    @pl.when(s < NS - 1)
    def _():
        pltpu.sync_copy(acc_sh.at[pl.ds(s * 632, 632)],
                        out_hbm.at[pl.ds(c * N + s * 632, 632)])

    @pl.when(s == NS - 1)
    def _():
        pltpu.sync_copy(acc_sh.at[pl.ds(15 * 632, 520)],
                        out_hbm.at[pl.ds(c * N + 15 * 632, 520)])


BR = 2000


def _tc_prescale_body(x_ref, w_ref, b_ref, d0_ref, d1_ref,
                      y_ref, r_ref, dis_ref):
    xw = jnp.dot(x_ref[...], w_ref[...], preferred_element_type=jnp.float32)
    degt = d0_ref[...] + d1_ref[...] + 1.0
    dis = lax.rsqrt(degt)
    y_ref[...] = xw * dis
    r_ref[...] = xw / degt + x_ref[...] + b_ref[...]
    dis_ref[...] = dis


def _tc_prescale(x, W, b2, deg2):
    grid = (N // BR,)
    return pl.pallas_call(
        _tc_prescale_body,
        grid=grid,
        in_specs=[
            pl.BlockSpec((BR, H), lambda i: (i, 0)),
            pl.BlockSpec((H, H), lambda i: (0, 0)),
            pl.BlockSpec((1, H), lambda i: (0, 0)),
            pl.BlockSpec((BR, 1), lambda i: (i, 0)),
            pl.BlockSpec((BR, 1), lambda i: (N // BR + i, 0)),
        ],
        out_specs=[
            pl.BlockSpec((BR, H), lambda i: (i, 0)),
            pl.BlockSpec((BR, H), lambda i: (i, 0)),
            pl.BlockSpec((BR, 1), lambda i: (i, 0)),
        ],
        out_shape=[
            jax.ShapeDtypeStruct((N, H), jnp.float32),
            jax.ShapeDtypeStruct((N, H), jnp.float32),
            jax.ShapeDtypeStruct((N, 1), jnp.float32),
        ],
    )(x, W, b2, deg2, deg2)


def _tc_finish_body(p0_ref, p1_ref, r_ref, dis_ref, o_ref):
    h = dis_ref[...] * (p0_ref[...] + p1_ref[...]) + r_ref[...]
    mean = jnp.mean(h, axis=1, keepdims=True)
    cent = h - mean
    var = jnp.mean(cent * cent, axis=1, keepdims=True)
    o_ref[...] = cent * lax.rsqrt(var + 1e-5)


def _tc_finish(acc, r, dis):
    grid = (N // BR,)
    return pl.pallas_call(
        _tc_finish_body,
        grid=grid,
        in_specs=[
            pl.BlockSpec((BR, H), lambda i: (i, 0)),
            pl.BlockSpec((BR, H), lambda i: (N // BR + i, 0)),
            pl.BlockSpec((BR, H), lambda i: (i, 0)),
            pl.BlockSpec((BR, 1), lambda i: (i, 0)),
        ],
        out_specs=pl.BlockSpec((BR, H), lambda i: (i, 0)),
        out_shape=jax.ShapeDtypeStruct((N, H), jnp.float32),
    )(acc, acc, r, dis)


def kernel(x, edge_index, batch, W, b):
    ei = edge_index.astype(jnp.int32).reshape(2 * E)

    ones_k = jnp.ones((KD,), jnp.float32)
    zeros_2k = jnp.zeros((2000,), jnp.float32)
    zrows = jnp.zeros((632, H), jnp.float32)

    deg = _sc_degree(ei, ones_k, zeros_2k)

    y, r, dis = _tc_prescale(x, W, b.reshape(1, H), deg.reshape(NC * N, 1))

    acc = _sc_scatter(y, ei, zrows)

    return _tc_finish(acc, r, dis)
